# Initial kernel scaffold; baseline (speedup 1.0000x reference)
import jax
import jax.numpy as jnp
from jax.experimental import pallas as pl


def kernel(x, edge_index, batch, W1, b1, W2, b2, Wl1, bl1, Wl2, bl2):
    raise NotImplementedError("write your pallas kernel here")


# speedup vs baseline: 10.8187x; 10.8187x over previous
import functools

import jax
import jax.numpy as jnp
from jax import lax
from jax.experimental import pallas as pl
from jax.experimental.pallas import tpu as pltpu
from jax.experimental.pallas import tpu_sc as plsc

N = 10000
E = 320000
C = 128
G = 64

NC = 2
NS = 16
CHUNK = 128
K = 79
EPAD = NC * NS * K * CHUNK
NACC = 10112
ZROWS = NACC // NS


def _sc_mesh():
    return plsc.VectorSubcoreMesh(core_axis_name="c", subcore_axis_name="s",
                                  num_cores=NC, num_subcores=NS)


def _deg_body(dst_hbm, ones_hbm, zeros_hbm, out_hbm, dst_v, ones_v, acc):
    c = lax.axis_index("c")
    s = lax.axis_index("s")
    wid = c * NS + s
    pltpu.sync_copy(ones_hbm, ones_v)
    pltpu.sync_copy(zeros_hbm, acc.at[pl.ds(s * ZROWS, ZROWS)])
    plsc.subcore_barrier()
    base = wid * K * CHUNK

    def body(j, carry):
        pltpu.sync_copy(dst_hbm.at[pl.ds(base + j * CHUNK, CHUNK)], dst_v)
        pltpu.sync_copy(ones_v, acc.at[dst_v], add=True)
        return carry

    lax.fori_loop(0, K, body, 0)
    plsc.subcore_barrier()
    pltpu.sync_copy(acc.at[pl.ds(s * ZROWS, ZROWS)],
                    out_hbm.at[c].at[pl.ds(s * ZROWS, ZROWS)])


@functools.cache
def _make_deg_pass():
    return pl.kernel(
        _deg_body,
        out_type=jax.ShapeDtypeStruct((NC, NACC, 16), jnp.float32),
        mesh=_sc_mesh(),
        scratch_types=[
            pltpu.VMEM((CHUNK,), jnp.int32),
            pltpu.VMEM((CHUNK, 16), jnp.float32),
            pltpu.VMEM_SHARED((NACC, 16), jnp.float32),
        ],
    )


def _edge_body(hp_hbm, src_hbm, dst_hbm, zeros_hbm, out_hbm,
               src_v, dst_v, rows_v, acc, sem):
    c = lax.axis_index("c")
    s = lax.axis_index("s")
    wid = c * NS + s
    pltpu.sync_copy(zeros_hbm, acc.at[pl.ds(s * ZROWS, ZROWS)])
    plsc.subcore_barrier()
    base = wid * K * CHUNK

    def body(j, carry):
        off = base + j * CHUNK
        pltpu.sync_copy(src_hbm.at[pl.ds(off, CHUNK)], src_v)
        pltpu.sync_copy(dst_hbm.at[pl.ds(off, CHUNK)], dst_v)
        pltpu.async_copy(hp_hbm.at[src_v], rows_v, sem).wait()
        pltpu.sync_copy(rows_v, acc.at[dst_v], add=True)
        return carry

    lax.fori_loop(0, K, body, 0)
    plsc.subcore_barrier()
    pltpu.sync_copy(acc.at[pl.ds(s * ZROWS, ZROWS)],
                    out_hbm.at[c].at[pl.ds(s * ZROWS, ZROWS)])


@functools.cache
def _make_edge_pass():
    return pl.kernel(
        _edge_body,
        out_type=jax.ShapeDtypeStruct((NC, NACC, C), jnp.float32),
        mesh=_sc_mesh(),
        scratch_types=[
            pltpu.VMEM((CHUNK,), jnp.int32),
            pltpu.VMEM((CHUNK,), jnp.int32),
            pltpu.VMEM((CHUNK, C), jnp.float32),
            pltpu.VMEM_SHARED((NACC, C), jnp.float32),
            pltpu.SemaphoreType.DMA,
        ],
    )


def _dense1_body(degp_ref, x_ref, w1_ref, hp1_ref):
    deg = degp_ref[0, :, 0] + degp_ref[1, :, 0] + 1.0
    dinv = lax.rsqrt(deg)
    h1 = jnp.dot(x_ref[...], w1_ref[...], preferred_element_type=jnp.float32)
    hp1_ref[...] = h1 * dinv[:, None]


_dense1 = pl.pallas_call(
    _dense1_body,
    out_shape=jax.ShapeDtypeStruct((NACC, C), jnp.float32),
)


def _dense2_body(s1_ref, hp1_ref, degp_ref, b1_ref, w2_ref, hp2_ref):
    deg = degp_ref[0, :, 0] + degp_ref[1, :, 0] + 1.0
    dinv = lax.rsqrt(deg)
    tot = s1_ref[0] + s1_ref[1] + hp1_ref[...]
    a1 = jnp.maximum(tot * dinv[:, None] + b1_ref[...], 0.0)
    h2 = jnp.dot(a1, w2_ref[...], preferred_element_type=jnp.float32)
    hp2_ref[...] = h2 * dinv[:, None]


_dense2 = pl.pallas_call(
    _dense2_body,
    out_shape=jax.ShapeDtypeStruct((NACC, C), jnp.float32),
)


def _dense3_body(s2_ref, hp2_ref, degp_ref, b2_ref, batch_ref,
                 wl1_ref, bl1_ref, wl2_ref, bl2_ref, out_ref):
    deg = degp_ref[0, :, 0] + degp_ref[1, :, 0] + 1.0
    dinv = lax.rsqrt(deg)
    tot = s2_ref[0] + s2_ref[1] + hp2_ref[...]
    h2 = jnp.maximum(tot * dinv[:, None] + b2_ref[...], 0.0)
    gids = lax.broadcasted_iota(jnp.int32, (NACC, G), 1)
    onehot = (batch_ref[...] == gids).astype(jnp.float32)
    sums = lax.dot_general(onehot, h2, (((0,), (0,)), ((), ())),
                           preferred_element_type=jnp.float32)
    counts = jnp.sum(onehot, axis=0)
    g = sums / jnp.maximum(counts, 1.0)[:, None]
    g = jnp.maximum(jnp.dot(g, wl1_ref[...],
                            preferred_element_type=jnp.float32)
                    + bl1_ref[...], 0.0)
    out_ref[...] = jnp.dot(g, wl2_ref[...],
                           preferred_element_type=jnp.float32) + bl2_ref[...]


_dense3 = pl.pallas_call(
    _dense3_body,
    out_shape=jax.ShapeDtypeStruct((G, 1), jnp.float32),
)


@jax.jit
def kernel(x, edge_index, batch, W1, b1, W2, b2, Wl1, bl1, Wl2, bl2):
    src = edge_index[0]
    dst = edge_index[1]
    npad = EPAD - E
    src_p = jnp.concatenate([src, jnp.zeros((npad,), jnp.int32)])
    dst_p = jnp.concatenate([dst, jnp.full((npad,), N, jnp.int32)])
    x_p = jnp.pad(x, ((0, NACC - N), (0, 0)))
    batch_p = jnp.concatenate([batch, jnp.full((NACC - N,), G, jnp.int32)])

    ones16 = jnp.ones((CHUNK, 16), jnp.float32)
    zeros16 = jnp.zeros((ZROWS, 16), jnp.float32)
    zerosC = jnp.zeros((ZROWS, C), jnp.float32)

    degp = _make_deg_pass()(dst_p, ones16, zeros16)

    hp1 = _dense1(degp, x_p, W1)
    s1 = _make_edge_pass()(hp1, src_p, dst_p, zerosC)

    hp2 = _dense2(s1, hp1, degp, b1.reshape(1, C), W2)
    s2 = _make_edge_pass()(hp2, src_p, dst_p, zerosC)

    out = _dense3(s2, hp2, degp, b2.reshape(1, C),
                  batch_p.reshape(NACC, 1),
                  Wl1, bl1.reshape(1, 32), Wl2, bl2.reshape(1, 1))
    return out.reshape(G)
